# baseline (speedup 1.0000x reference)
import functools

import jax
import jax.numpy as jnp
from jax import lax
from jax.experimental import pallas as pl
from jax.experimental.pallas import tpu as pltpu
from jax.experimental.pallas import tpu_sc as plsc

BATCH = 16384
VOCAB = 1000
VPAD = 1024
L = 16
NC, NS = 2, 16
NW = NC * NS
CHUNK = BATCH // NW


def _lr_body(uid_hbm, iid_hbm, hs_u_hbm, hs_i_hbm, bu_hbm, cu_hbm,
             bi_hbm, ci_hbm, wu_hbm, wi_hbm, out_hbm,
             hs_v, b_v, c_v, w_v, fused_v, tab_v,
             uid_v, iid_v, out_v, shared, sem, sem_ids):
    sid = lax.axis_index("s")
    cid = lax.axis_index("c")
    wid = sid * NC + cid
    base = wid * CHUNK

    id_copies = [
        pltpu.async_copy(uid_hbm.at[pl.ds(base, CHUNK)], uid_v, sem_ids),
        pltpu.async_copy(iid_hbm.at[pl.ds(base, CHUNK)], iid_v, sem_ids),
    ]

    SEG = 512

    def fuse_segment(hs_hbm, b_hbm, c_hbm, w_hbm, row, off, n):
        copies = [
            pltpu.async_copy(hs_hbm.at[pl.ds(off, n)], hs_v.at[pl.ds(0, n)], sem),
            pltpu.async_copy(b_hbm.at[pl.ds(off, n)], b_v.at[pl.ds(0, n)], sem),
            pltpu.async_copy(c_hbm.at[pl.ds(off, n)], c_v.at[pl.ds(0, n)], sem),
            pltpu.async_copy(w_hbm.at[pl.ds(off, n)], w_v.at[pl.ds(0, n)], sem),
        ]
        for cp in copies:
            cp.wait()

        @plsc.parallel_loop(0, SEG, step=L, unroll=4)
        def _(j):
            s = pl.ds(j, L)
            fused_v[s] = w_v[s] * (b_v[s] * hs_v[s] + c_v[s])

        pltpu.sync_copy(fused_v, shared.at[pl.ds(row * VPAD + off, SEG)])

    user_tabs = (hs_u_hbm, bu_hbm, cu_hbm, wu_hbm)
    item_tabs = (hs_i_hbm, bi_hbm, ci_hbm, wi_hbm)
    for k in range(2):
        off = k * SEG
        n = min(SEG, VOCAB - off)

        @pl.when(sid == k)
        def _(off=off, n=n):
            fuse_segment(*user_tabs, 0, off, n)

        @pl.when(sid == 2 + k)
        def _(off=off, n=n):
            fuse_segment(*item_tabs, 1, off, n)

    plsc.subcore_barrier()

    tc_u = pltpu.async_copy(shared.at[pl.ds(0, VPAD)],
                            tab_v.at[pl.ds(0, VPAD)], sem)
    tc_i = pltpu.async_copy(shared.at[pl.ds(VPAD, VPAD)],
                            tab_v.at[pl.ds(VPAD, VPAD)], sem_ids)
    tc_u.wait()
    for cp in id_copies:
        cp.wait()

    @plsc.parallel_loop(0, CHUNK, step=L, unroll=4)
    def _(i):
        s = pl.ds(i, L)
        out_v[s] = plsc.load_gather(tab_v, [uid_v[s] - 1])

    tc_i.wait()

    @plsc.parallel_loop(0, CHUNK, step=L, unroll=4)
    def _(i):
        s = pl.ds(i, L)
        out_v[s] = out_v[s] + plsc.load_gather(tab_v, [iid_v[s] + (VPAD - 1)])

    pltpu.sync_copy(out_v, out_hbm.at[pl.ds(base, CHUNK)])


@functools.partial(
    pl.kernel,
    out_type=jax.ShapeDtypeStruct((BATCH,), jnp.float32),
    mesh=plsc.VectorSubcoreMesh(core_axis_name="c", subcore_axis_name="s"),
    compiler_params=pltpu.CompilerParams(needs_layout_passes=False),
    scratch_types=[pltpu.VMEM((512,), jnp.float32) for _ in range(5)]
    + [pltpu.VMEM((2 * VPAD,), jnp.float32)]
    + [pltpu.VMEM((CHUNK,), jnp.int32) for _ in range(2)]
    + [pltpu.VMEM((CHUNK,), jnp.float32),
       pltpu.VMEM_SHARED((2 * VPAD,), jnp.float32),
       pltpu.SemaphoreType.DMA, pltpu.SemaphoreType.DMA],
)
def _lr_kernel(*refs):
    _lr_body(*refs)


def kernel(user_id, item_id, user_hs, item_hs, beta_u, bias_u,
           beta_i, bias_i, user_weight, item_weight):
    out = _lr_kernel(user_id, item_id, user_hs.reshape(-1), item_hs.reshape(-1),
                     beta_u.reshape(-1), bias_u.reshape(-1),
                     beta_i.reshape(-1), bias_i.reshape(-1),
                     user_weight.reshape(-1), item_weight.reshape(-1))
    return out.reshape(BATCH, 1)


# speedup vs baseline: 1.0081x; 1.0081x over previous
import functools

import jax
import jax.numpy as jnp
from jax import lax
from jax.experimental import pallas as pl
from jax.experimental.pallas import tpu as pltpu
from jax.experimental.pallas import tpu_sc as plsc

BATCH = 16384
VOCAB = 1000
VPAD = 1024
L = 16
NC, NS = 2, 16
NW = NC * NS
CHUNK = BATCH // NW


def _lr_body(uid_hbm, iid_hbm, hs_u_hbm, hs_i_hbm, bu_hbm, cu_hbm,
             bi_hbm, ci_hbm, wu_hbm, wi_hbm, out_hbm,
             hs_v, b_v, c_v, w_v, fused_v, tab_v,
             uid_v, iid_v, out_v, shared, sem, sem_ids):
    sid = lax.axis_index("s")
    cid = lax.axis_index("c")
    wid = sid * NC + cid
    base = wid * CHUNK

    id_copies = [
        pltpu.async_copy(uid_hbm.at[pl.ds(base, CHUNK)], uid_v, sem_ids),
        pltpu.async_copy(iid_hbm.at[pl.ds(base, CHUNK)], iid_v, sem_ids),
    ]

    SEG = 512

    def fuse_segment(hs_hbm, b_hbm, c_hbm, w_hbm, row, off, n):
        copies = [
            pltpu.async_copy(hs_hbm.at[pl.ds(off, n)], hs_v.at[pl.ds(0, n)], sem),
            pltpu.async_copy(b_hbm.at[pl.ds(off, n)], b_v.at[pl.ds(0, n)], sem),
            pltpu.async_copy(c_hbm.at[pl.ds(off, n)], c_v.at[pl.ds(0, n)], sem),
            pltpu.async_copy(w_hbm.at[pl.ds(off, n)], w_v.at[pl.ds(0, n)], sem),
        ]
        for cp in copies:
            cp.wait()

        @plsc.parallel_loop(0, SEG, step=L, unroll=4)
        def _(j):
            s = pl.ds(j, L)
            fused_v[s] = w_v[s] * (b_v[s] * hs_v[s] + c_v[s])

        pltpu.sync_copy(fused_v, shared.at[pl.ds(row * VPAD + off, SEG)])

    user_tabs = (hs_u_hbm, bu_hbm, cu_hbm, wu_hbm)
    item_tabs = (hs_i_hbm, bi_hbm, ci_hbm, wi_hbm)
    for k in range(2):
        off = k * SEG
        n = min(SEG, VOCAB - off)

        @pl.when(sid == k)
        def _(off=off, n=n):
            fuse_segment(*user_tabs, 0, off, n)

        @pl.when(sid == 2 + k)
        def _(off=off, n=n):
            fuse_segment(*item_tabs, 1, off, n)

    plsc.subcore_barrier()

    pltpu.async_copy(shared, tab_v, sem).wait()
    for cp in id_copies:
        cp.wait()

    @plsc.parallel_loop(0, CHUNK, step=L, unroll=4)
    def _(i):
        s = pl.ds(i, L)
        out_v[s] = (plsc.load_gather(tab_v, [uid_v[s] - 1])
                    + plsc.load_gather(tab_v, [iid_v[s] + (VPAD - 1)]))

    pltpu.sync_copy(out_v, out_hbm.at[pl.ds(base, CHUNK)])


@functools.partial(
    pl.kernel,
    out_type=jax.ShapeDtypeStruct((BATCH,), jnp.float32),
    mesh=plsc.VectorSubcoreMesh(core_axis_name="c", subcore_axis_name="s"),
    compiler_params=pltpu.CompilerParams(needs_layout_passes=False),
    scratch_types=[pltpu.VMEM((512,), jnp.float32) for _ in range(5)]
    + [pltpu.VMEM((2 * VPAD,), jnp.float32)]
    + [pltpu.VMEM((CHUNK,), jnp.int32) for _ in range(2)]
    + [pltpu.VMEM((CHUNK,), jnp.float32),
       pltpu.VMEM_SHARED((2 * VPAD,), jnp.float32),
       pltpu.SemaphoreType.DMA, pltpu.SemaphoreType.DMA],
)
def _lr_kernel(*refs):
    _lr_body(*refs)


def kernel(user_id, item_id, user_hs, item_hs, beta_u, bias_u,
           beta_i, bias_i, user_weight, item_weight):
    out = _lr_kernel(user_id, item_id, user_hs.reshape(-1), item_hs.reshape(-1),
                     beta_u.reshape(-1), bias_u.reshape(-1),
                     beta_i.reshape(-1), bias_i.reshape(-1),
                     user_weight.reshape(-1), item_weight.reshape(-1))
    return out.reshape(BATCH, 1)
